# Initial kernel scaffold; baseline (speedup 1.0000x reference)
import jax
import jax.numpy as jnp
from jax.experimental import pallas as pl


def kernel(h_dense, Wq, bq, Wk, bk, Wv, bv, We, be):
    raise NotImplementedError("write your pallas kernel here")


# speedup vs baseline: 5.5585x; 5.5585x over previous
import functools

import jax
import jax.numpy as jnp
from jax.experimental import pallas as pl
from jax.experimental.pallas import tpu as pltpu

_HI = jax.lax.Precision.HIGHEST


def _qk_kernel(wq_ref, bq_ref, wk_ref, qk_ref):
    i = pl.program_id(0)
    qc = jnp.sum(wq_ref[...], axis=1) + bq_ref[0, :]
    part = jax.lax.dot_general(
        qc[None, :], wk_ref[...], (((1,), (0,)), ((), ())),
        preferred_element_type=jnp.float32, precision=_HI)

    @pl.when(i == 0)
    def _():
        qk_ref[...] = jnp.zeros_like(qk_ref)

    qk_ref[...] += part


def _flash_kernel(h_ref, qk_ref, hbar_ref, acc_ref, m_ref, s_ref, *, inv_scale):
    i = pl.program_id(0)
    nsteps = pl.num_programs(0)

    @pl.when(i == 0)
    def _():
        m_ref[...] = jnp.full_like(m_ref, -jnp.inf)
        s_ref[...] = jnp.zeros_like(s_ref)
        acc_ref[...] = jnp.zeros_like(acc_ref)

    h = h_ref[...]
    qk = qk_ref[0, :]
    t = jax.lax.dot_general(
        h, qk, (((2,), (0,)), ((), ())),
        preferred_element_type=jnp.float32, precision=_HI)
    t = t * inv_scale

    m_prev = m_ref[...]
    m_new = jnp.maximum(m_prev, jnp.max(t, axis=1, keepdims=True))
    alpha = jnp.exp(m_prev - m_new)
    p = jnp.exp(t - m_new)
    s_ref[...] = s_ref[...] * alpha + jnp.sum(p, axis=1, keepdims=True)
    pv = jax.lax.dot_general(
        p, h, (((1,), (1,)), ((0,), (0,))),
        preferred_element_type=jnp.float32, precision=_HI)
    acc_ref[...] = acc_ref[...] * alpha + pv
    m_ref[...] = m_new

    @pl.when(i == nsteps - 1)
    def _():
        hbar_ref[...] = acc_ref[...] / s_ref[...]


def _tail_kernel(hbar_ref, wv_ref, bv_ref, we_ref, be_ref,
                 expert_ref, pmax_ref, logits_ref):
    i = pl.program_id(0)
    nsteps = pl.num_programs(0)
    r = jax.lax.dot_general(
        hbar_ref[...], wv_ref[...], (((1,), (1,)), ((), ())),
        preferred_element_type=jnp.float32, precision=_HI)
    r = r + bv_ref[...]
    part = jax.lax.dot_general(
        r, we_ref[...], (((1,), (1,)), ((), ())),
        preferred_element_type=jnp.float32, precision=_HI)

    @pl.when(i == 0)
    def _():
        logits_ref[...] = jnp.zeros_like(logits_ref)

    logits_ref[...] += part

    @pl.when(i == nsteps - 1)
    def _():
        logits = logits_ref[...] + be_ref[...]
        logits_ref[...] = logits
        row_max = jnp.max(logits, axis=1, keepdims=True)
        ex = jnp.exp(logits - row_max)
        denom = jnp.sum(ex, axis=1, keepdims=True)
        pmax_ref[...] = jnp.max(ex, axis=1, keepdims=True) / denom
        bdim, edim = logits.shape
        idx = jax.lax.broadcasted_iota(jnp.int32, (bdim, edim), 1)
        am = jnp.min(jnp.where(logits == row_max, idx, edim),
                     axis=1, keepdims=True)
        expert_ref[...] = (idx == am).astype(jnp.int32)


def kernel(h_dense, Wq, bq, Wk, bk, Wv, bv, We, be):
    del bk
    B, N, D = h_dense.shape
    E = We.shape[0]
    f32 = jnp.float32

    C1 = 512
    qk = pl.pallas_call(
        _qk_kernel,
        grid=(D // C1,),
        in_specs=[
            pl.BlockSpec((C1, D), lambda i: (i, 0)),
            pl.BlockSpec((1, C1), lambda i: (0, i)),
            pl.BlockSpec((C1, D), lambda i: (i, 0)),
        ],
        out_specs=pl.BlockSpec((1, D), lambda i: (0, 0)),
        out_shape=jax.ShapeDtypeStruct((1, D), f32),
    )(Wq, bq.reshape(1, D), Wk)

    C2 = 256
    hbar = pl.pallas_call(
        functools.partial(_flash_kernel, inv_scale=1.0 / (float(D) ** 0.5)),
        grid=(N // C2,),
        in_specs=[
            pl.BlockSpec((B, C2, D), lambda i: (0, i, 0)),
            pl.BlockSpec((1, D), lambda i: (0, 0)),
        ],
        out_specs=pl.BlockSpec((B, D), lambda i: (0, 0)),
        out_shape=jax.ShapeDtypeStruct((B, D), f32),
        scratch_shapes=[
            pltpu.VMEM((B, D), f32),
            pltpu.VMEM((B, 1), f32),
            pltpu.VMEM((B, 1), f32),
        ],
    )(h_dense, qk)

    C3 = 512
    expert, pmax, logits = pl.pallas_call(
        _tail_kernel,
        grid=(D // C3,),
        in_specs=[
            pl.BlockSpec((B, D), lambda i: (0, 0)),
            pl.BlockSpec((C3, D), lambda i: (i, 0)),
            pl.BlockSpec((1, C3), lambda i: (0, i)),
            pl.BlockSpec((E, C3), lambda i: (0, i)),
            pl.BlockSpec((1, E), lambda i: (0, 0)),
        ],
        out_specs=[
            pl.BlockSpec((B, E), lambda i: (0, 0)),
            pl.BlockSpec((B, 1), lambda i: (0, 0)),
            pl.BlockSpec((B, E), lambda i: (0, 0)),
        ],
        out_shape=[
            jax.ShapeDtypeStruct((B, E), jnp.int32),
            jax.ShapeDtypeStruct((B, 1), f32),
            jax.ShapeDtypeStruct((B, E), f32),
        ],
    )(hbar, Wv, bv.reshape(1, D), We, be.reshape(1, E))

    return (expert, pmax, logits)


# speedup vs baseline: 9.0803x; 1.6336x over previous
import functools

import jax
import jax.numpy as jnp
from jax.experimental import pallas as pl
from jax.experimental.pallas import tpu as pltpu

_HI = jax.lax.Precision.HIGHEST


def _fused_kernel(wq_ref, bq_ref, wk_ref, h_ref, wv_ref, bv_ref, we_ref, be_ref,
                  expert_ref, pmax_ref, logits_ref,
                  qk_ref, hbar_ref, acc_ref, m_ref, s_ref,
                  *, s1, s2, s3, inv_scale):
    i = pl.program_id(0)

    @pl.when(i < s1)
    def _phase1():
        qc = jnp.sum(wq_ref[...], axis=1) + bq_ref[0, :]
        part = jnp.sum(qc[:, None] * wk_ref[...], axis=0, keepdims=True)

        @pl.when(i == 0)
        def _():
            qk_ref[...] = jnp.zeros_like(qk_ref)

        qk_ref[...] += part

    @pl.when((i >= s1) & (i < s1 + s2))
    def _phase2():
        @pl.when(i == s1)
        def _():
            m_ref[...] = jnp.full_like(m_ref, -jnp.inf)
            s_ref[...] = jnp.zeros_like(s_ref)
            acc_ref[...] = jnp.zeros_like(acc_ref)

        h = h_ref[...]
        t = jax.lax.dot_general(
            h, qk_ref[0, :], (((2,), (0,)), ((), ())),
            preferred_element_type=jnp.float32, precision=_HI)
        t = t * inv_scale
        m_prev = m_ref[...]
        m_new = jnp.maximum(m_prev, jnp.max(t, axis=1, keepdims=True))
        alpha = jnp.exp(m_prev - m_new)
        p = jnp.exp(t - m_new)
        s_ref[...] = s_ref[...] * alpha + jnp.sum(p, axis=1, keepdims=True)
        pv = jax.lax.dot_general(
            p, h, (((1,), (1,)), ((0,), (0,))),
            preferred_element_type=jnp.float32)
        acc_ref[...] = acc_ref[...] * alpha + pv
        m_ref[...] = m_new

        @pl.when(i == s1 + s2 - 1)
        def _():
            hbar_ref[...] = acc_ref[...] / s_ref[...]

    @pl.when(i >= s1 + s2)
    def _phase3():
        hb = hbar_ref[...]
        hb_hi = hb.astype(jnp.bfloat16)
        hb_lo = (hb - hb_hi.astype(jnp.float32)).astype(jnp.bfloat16)
        hb2 = jnp.concatenate([hb_hi, hb_lo], axis=0)
        wv_hi = wv_ref[...].astype(jnp.bfloat16)
        bdim = hb.shape[0]
        rr = jax.lax.dot_general(
            hb2, wv_hi, (((1,), (1,)), ((), ())),
            preferred_element_type=jnp.float32)
        r = rr[:bdim, :] + rr[bdim:, :] + bv_ref[...]
        part = jax.lax.dot_general(
            r, we_ref[...], (((1,), (1,)), ((), ())),
            preferred_element_type=jnp.float32, precision=_HI)

        @pl.when(i == s1 + s2)
        def _():
            logits_ref[...] = jnp.zeros_like(logits_ref)

        logits_ref[...] += part

        @pl.when(i == s1 + s2 + s3 - 1)
        def _():
            logits = logits_ref[...] + be_ref[...]
            logits_ref[...] = logits
            row_max = jnp.max(logits, axis=1, keepdims=True)
            ex = jnp.exp(logits - row_max)
            denom = jnp.sum(ex, axis=1, keepdims=True)
            pmax_ref[...] = jnp.max(ex, axis=1, keepdims=True) / denom
            bd, ed = logits.shape
            idx = jax.lax.broadcasted_iota(jnp.int32, (bd, ed), 1)
            am = jnp.min(jnp.where(logits == row_max, idx, ed),
                         axis=1, keepdims=True)
            expert_ref[...] = (idx == am).astype(jnp.int32)


def kernel(h_dense, Wq, bq, Wk, bk, Wv, bv, We, be):
    del bk
    B, N, D = h_dense.shape
    E = We.shape[0]
    f32 = jnp.float32

    C1, C2, C3 = 256, 256, 256
    s1, s2, s3 = D // C1, N // C2, D // C3
    last1, last2, last3 = s1 - 1, s2 - 1, s3 - 1

    def _clip(v, hi):
        return jnp.minimum(jnp.maximum(v, 0), hi)

    expert, pmax, logits = pl.pallas_call(
        functools.partial(_fused_kernel, s1=s1, s2=s2, s3=s3,
                          inv_scale=1.0 / (float(D) ** 0.5)),
        grid=(s1 + s2 + s3,),
        in_specs=[
            pl.BlockSpec((C1, D), lambda i: (_clip(i, last1), 0)),
            pl.BlockSpec((1, C1), lambda i: (0, _clip(i, last1))),
            pl.BlockSpec((C1, D), lambda i: (_clip(i, last1), 0)),
            pl.BlockSpec((B, C2, D), lambda i: (0, _clip(i - s1, last2), 0)),
            pl.BlockSpec((C3, D), lambda i: (_clip(i - s1 - s2, last3), 0)),
            pl.BlockSpec((1, C3), lambda i: (0, _clip(i - s1 - s2, last3))),
            pl.BlockSpec((E, C3), lambda i: (0, _clip(i - s1 - s2, last3))),
            pl.BlockSpec((1, E), lambda i: (0, 0)),
        ],
        out_specs=[
            pl.BlockSpec((B, E), lambda i: (0, 0)),
            pl.BlockSpec((B, 1), lambda i: (0, 0)),
            pl.BlockSpec((B, E), lambda i: (0, 0)),
        ],
        out_shape=[
            jax.ShapeDtypeStruct((B, E), jnp.int32),
            jax.ShapeDtypeStruct((B, 1), f32),
            jax.ShapeDtypeStruct((B, E), f32),
        ],
        scratch_shapes=[
            pltpu.VMEM((1, D), f32),
            pltpu.VMEM((B, D), f32),
            pltpu.VMEM((B, D), f32),
            pltpu.VMEM((B, 1), f32),
            pltpu.VMEM((B, 1), f32),
        ],
    )(Wq, bq.reshape(1, D), Wk, h_dense, Wv, bv.reshape(1, D), We,
      be.reshape(1, E))

    return (expert, pmax, logits)


# speedup vs baseline: 9.1135x; 1.0037x over previous
import functools

import jax
import jax.numpy as jnp
from jax.experimental import pallas as pl
from jax.experimental.pallas import tpu as pltpu

_HI = jax.lax.Precision.HIGHEST

_NS = 2
_C1 = 256
_C2 = 256
_C3 = 256


def _fused_kernel(*refs, s1, s2, s3, ns, dh, inv_scale):
    (wq, bq, wk, h, wv, bv, we, be,
     expert_ref, pmax_ref, logits_ref,
     qk_ref, hbar_ref, acc_ref, m_ref, s_ref) = refs
    wq, wk, h, wv = list(wq), list(wk), list(h), list(wv)
    i = pl.program_id(0)

    @pl.when(i < s1)
    def _phase1():
        qc = bq[0, :]
        for j in range(ns):
            qc = qc + jnp.sum(wq[j][...], axis=1)

        @pl.when(i == 0)
        def _():
            qk_ref[...] = jnp.zeros_like(qk_ref)

        for j in range(ns):
            part = jnp.sum(qc[:, None] * wk[j][...], axis=0, keepdims=True)
            qk_ref[:, j * dh:(j + 1) * dh] += part

    @pl.when((i >= s1) & (i < s1 + s2))
    def _phase2():
        @pl.when(i == s1)
        def _():
            m_ref[...] = jnp.full_like(m_ref, -jnp.inf)
            s_ref[...] = jnp.zeros_like(s_ref)
            acc_ref[...] = jnp.zeros_like(acc_ref)

        hs = [h[j][...] for j in range(ns)]
        t = None
        for j in range(ns):
            tj = jax.lax.dot_general(
                hs[j], qk_ref[0, j * dh:(j + 1) * dh],
                (((2,), (0,)), ((), ())),
                preferred_element_type=jnp.float32, precision=_HI)
            t = tj if t is None else t + tj
        t = t * inv_scale
        m_prev = m_ref[...]
        m_new = jnp.maximum(m_prev, jnp.max(t, axis=1, keepdims=True))
        alpha = jnp.exp(m_prev - m_new)
        p = jnp.exp(t - m_new)
        s_ref[...] = s_ref[...] * alpha + jnp.sum(p, axis=1, keepdims=True)
        for j in range(ns):
            pvj = jax.lax.dot_general(
                p, hs[j], (((1,), (1,)), ((0,), (0,))),
                preferred_element_type=jnp.float32)
            sl = slice(j * dh, (j + 1) * dh)
            acc_ref[:, sl] = acc_ref[:, sl] * alpha + pvj
        m_ref[...] = m_new

        @pl.when(i == s1 + s2 - 1)
        def _():
            hbar_ref[...] = acc_ref[...] / s_ref[...]

    @pl.when(i >= s1 + s2)
    def _phase3():
        r = bv[...]
        bdim = hbar_ref.shape[0]
        for j in range(ns):
            hb = hbar_ref[:, j * dh:(j + 1) * dh]
            hb_hi = hb.astype(jnp.bfloat16)
            hb_lo = (hb - hb_hi.astype(jnp.float32)).astype(jnp.bfloat16)
            hb2 = jnp.concatenate([hb_hi, hb_lo], axis=0)
            wv_hi = wv[j][...].astype(jnp.bfloat16)
            rr = jax.lax.dot_general(
                hb2, wv_hi, (((1,), (1,)), ((), ())),
                preferred_element_type=jnp.float32)
            r = r + rr[:bdim, :] + rr[bdim:, :]
        part = jax.lax.dot_general(
            r, we[...], (((1,), (1,)), ((), ())),
            preferred_element_type=jnp.float32, precision=_HI)

        @pl.when(i == s1 + s2)
        def _():
            logits_ref[...] = jnp.zeros_like(logits_ref)

        logits_ref[...] += part

        @pl.when(i == s1 + s2 + s3 - 1)
        def _():
            logits = logits_ref[...] + be[...]
            logits_ref[...] = logits
            row_max = jnp.max(logits, axis=1, keepdims=True)
            ex = jnp.exp(logits - row_max)
            denom = jnp.sum(ex, axis=1, keepdims=True)
            pmax_ref[...] = jnp.max(ex, axis=1, keepdims=True) / denom
            bd, ed = logits.shape
            idx = jax.lax.broadcasted_iota(jnp.int32, (bd, ed), 1)
            am = jnp.min(jnp.where(logits == row_max, idx, ed),
                         axis=1, keepdims=True)
            expert_ref[...] = (idx == am).astype(jnp.int32)


def kernel(h_dense, Wq, bq, Wk, bk, Wv, bv, We, be):
    del bk
    B, N, D = h_dense.shape
    E = We.shape[0]
    f32 = jnp.float32

    ns = _NS
    dh = D // ns
    s1, s2, s3 = D // _C1, N // _C2, D // _C3
    l1, l2, l3 = s1 - 1, s2 - 1, s3 - 1

    def _clip(v, hi):
        return jnp.minimum(jnp.maximum(v, 0), hi)

    def _wq_spec(j):
        return pl.BlockSpec((_C1, dh), lambda i: (_clip(i, l1), j))

    def _wk_spec(j):
        return pl.BlockSpec((_C1, dh), lambda i: (_clip(i, l1), j))

    def _h_spec(j):
        return pl.BlockSpec((B, _C2, dh), lambda i: (0, _clip(i - s1, l2), j))

    def _wv_spec(j):
        return pl.BlockSpec((_C3, dh), lambda i: (_clip(i - s1 - s2, l3), j))

    in_specs = (
        [_wq_spec(j) for j in range(ns)]
        + [pl.BlockSpec((1, _C1), lambda i: (0, _clip(i, l1)))]
        + [_wk_spec(j) for j in range(ns)]
        + [_h_spec(j) for j in range(ns)]
        + [_wv_spec(j) for j in range(ns)]
        + [pl.BlockSpec((1, _C3), lambda i: (0, _clip(i - s1 - s2, l3))),
           pl.BlockSpec((E, _C3), lambda i: (0, _clip(i - s1 - s2, l3))),
           pl.BlockSpec((1, E), lambda i: (0, 0))]
    )
    operands = ([Wq] * ns + [bq.reshape(1, D)] + [Wk] * ns + [h_dense] * ns
                + [Wv] * ns + [bv.reshape(1, D), We, be.reshape(1, E)])

    def _body(*refs):
        wq = refs[0:ns]
        bq_r = refs[ns]
        wk = refs[ns + 1:2 * ns + 1]
        h = refs[2 * ns + 1:3 * ns + 1]
        wv = refs[3 * ns + 1:4 * ns + 1]
        bv_r, we_r, be_r = refs[4 * ns + 1:4 * ns + 4]
        rest = refs[4 * ns + 4:]
        return _fused_kernel(wq, bq_r, wk, h, wv, bv_r, we_r, be_r, *rest,
                             s1=s1, s2=s2, s3=s3, ns=ns, dh=dh,
                             inv_scale=1.0 / (float(D) ** 0.5))

    expert, pmax, logits = pl.pallas_call(
        _body,
        grid=(s1 + s2 + s3,),
        in_specs=in_specs,
        out_specs=[
            pl.BlockSpec((B, E), lambda i: (0, 0)),
            pl.BlockSpec((B, 1), lambda i: (0, 0)),
            pl.BlockSpec((B, E), lambda i: (0, 0)),
        ],
        out_shape=[
            jax.ShapeDtypeStruct((B, E), jnp.int32),
            jax.ShapeDtypeStruct((B, 1), f32),
            jax.ShapeDtypeStruct((B, E), f32),
        ],
        scratch_shapes=[
            pltpu.VMEM((1, D), f32),
            pltpu.VMEM((B, D), f32),
            pltpu.VMEM((B, D), f32),
            pltpu.VMEM((B, 1), f32),
            pltpu.VMEM((B, 1), f32),
        ],
    )(*operands)

    return (expert, pmax, logits)
